# baseline (speedup 1.0000x reference)
import functools

import jax
import jax.numpy as jnp
from jax import lax
from jax.experimental import pallas as pl
from jax.experimental.pallas import tpu as pltpu
from jax.experimental.pallas import tpu_sc as plsc

_NC = 2
_NS = 16
_NW = _NC * _NS
_C = 4


def kernel(x, emb):
    (B,) = x.shape
    V, D = emb.shape
    bpw = B // _NW
    nchunk = bpw // _C

    x3 = x.reshape(_NW, nchunk, _C).astype(jnp.int32)

    mesh = plsc.VectorSubcoreMesh(core_axis_name="c", subcore_axis_name="s")

    @functools.partial(
        pl.kernel,
        out_type=jax.ShapeDtypeStruct((B // _C, _C, D), emb.dtype),
        mesh=mesh,
        scratch_types=[
            pltpu.VMEM((nchunk, _C), jnp.int32),
            pltpu.VMEM((_C, D), emb.dtype),
            pltpu.VMEM((_C, D), emb.dtype),
            pltpu.SemaphoreType.DMA,
            pltpu.SemaphoreType.DMA,
        ],
    )
    def gather_k(x_hbm, emb_hbm, out_hbm, idx_v, buf0, buf1, sem0, sem1):
        wid = lax.axis_index("s") * _NC + lax.axis_index("c")
        cbase = wid * nchunk
        pltpu.sync_copy(x_hbm.at[wid], idx_v)

        pltpu.async_copy(emb_hbm.at[idx_v.at[0]], buf0, sem0)

        @pl.loop(0, nchunk, step=2)
        def _(j):
            pltpu.async_copy(emb_hbm.at[idx_v.at[j + 1]], buf1, sem1)
            pltpu.make_async_copy(emb_hbm.at[idx_v.at[j]], buf0, sem0).wait()
            pltpu.sync_copy(buf0, out_hbm.at[cbase + j])

            @pl.when(j + 2 < nchunk)
            def _():
                pltpu.async_copy(emb_hbm.at[idx_v.at[j + 2]], buf0, sem0)

            pltpu.make_async_copy(emb_hbm.at[idx_v.at[j + 1]], buf1, sem1).wait()
            pltpu.sync_copy(buf1, out_hbm.at[cbase + j + 1])

    out = gather_k(x3, emb)
    return out.reshape(B, D)
